# Initial kernel scaffold; baseline (speedup 1.0000x reference)
import jax
import jax.numpy as jnp
from jax.experimental import pallas as pl


def kernel(x, edge_index, W1, b1, W2, b2):
    raise NotImplementedError("write your pallas kernel here")


# speedup vs baseline: 11.0645x; 11.0645x over previous
import functools

import jax
import jax.numpy as jnp
from jax import lax
from jax.experimental import pallas as pl
from jax.experimental.pallas import tpu as pltpu
from jax.experimental.pallas import tpu_sc as plsc

_N = 10000
_NPAD = 10240
_NC = 2
_NS = 16
_L = 16
_NW = _NC * _NS
_K = 128
_RPT = _NPAD // _NS


def _sc_mesh():
    return plsc.VectorSubcoreMesh(
        core_axis_name="c", subcore_axis_name="s",
        num_cores=_NC, num_subcores=_NS)


def _sc_degree(dst3):
    cpt = dst3.shape[1]

    @functools.partial(
        pl.kernel,
        out_type=[jax.ShapeDtypeStruct((_NPAD, _L), jnp.float32),
                  jax.ShapeDtypeStruct((_NPAD, _L), jnp.float32)],
        mesh=_sc_mesh(),
        scratch_types=[
            pltpu.VMEM((cpt, _K), jnp.int32),
            pltpu.VMEM((_K, _L), jnp.float32),
            pltpu.VMEM((_K, _L), jnp.float32),
            pltpu.VMEM_SHARED((_NPAD, _L), jnp.float32),
        ],
    )
    def deg_kernel(dst_hbm, d0_hbm, d1_hbm, idx_v, ones_v, zb_v, hist):
        c = lax.axis_index("c")
        s = lax.axis_index("s")
        w = c * _NS + s

        @pl.loop(0, _K)
        def _(i):
            ones_v[i] = jnp.full((_L,), 1.0, jnp.float32)
            zb_v[i] = jnp.zeros((_L,), jnp.float32)

        for r in range(_RPT // _K):
            pltpu.sync_copy(zb_v, hist.at[pl.ds(s * _RPT + r * _K, _K)])
        plsc.subcore_barrier()

        pltpu.sync_copy(dst_hbm.at[w], idx_v)

        @pl.loop(0, cpt)
        def _(j):
            pltpu.sync_copy(ones_v, hist.at[idx_v.at[j]], add=True)

        plsc.subcore_barrier()
        sl = pl.ds(s * _RPT, _RPT)

        @pl.when(c == 0)
        def _():
            pltpu.sync_copy(hist.at[sl], d0_hbm.at[sl])

        @pl.when(c == 1)
        def _():
            pltpu.sync_copy(hist.at[sl], d1_hbm.at[sl])

    return deg_kernel(dst3)


def _sc_agg_split(src3, dst3, u0, u1):
    cpt = src3.shape[1]

    @functools.partial(
        pl.kernel,
        out_type=[jax.ShapeDtypeStruct((_NPAD, 128), jnp.float32),
                  jax.ShapeDtypeStruct((_NPAD, 128), jnp.float32)],
        mesh=_sc_mesh(),
        scratch_types=[
            pltpu.VMEM((cpt, _K), jnp.int32),
            pltpu.VMEM((cpt, _K), jnp.int32),
            pltpu.VMEM((_K, 128), jnp.float32),
            pltpu.VMEM_SHARED((_NPAD, 128), jnp.float32),
            pltpu.SemaphoreType.DMA,
        ],
    )
    def agg_kernel(src_hbm, dst_hbm, u0_hbm, u1_hbm, o0_hbm, o1_hbm,
                   src_v, dst_v, buf, acc, sem):
        c = lax.axis_index("c")
        s = lax.axis_index("s")

        @pl.loop(0, _K)
        def _(i):
            @pl.loop(0, 128, step=_L)
            def _(j):
                buf[i, pl.ds(j, _L)] = jnp.zeros((_L,), jnp.float32)

        for r in range(_RPT // _K):
            pltpu.sync_copy(buf, acc.at[pl.ds(s * _RPT + r * _K, _K)])
        plsc.subcore_barrier()

        pltpu.sync_copy(src_hbm.at[s], src_v)
        pltpu.sync_copy(dst_hbm.at[s], dst_v)

        def run(u_hbm):
            @pl.loop(0, cpt)
            def _(j):
                pltpu.async_copy(u_hbm.at[src_v.at[j]], buf, sem).wait()
                pltpu.sync_copy(buf, acc.at[dst_v.at[j]], add=True)

        @pl.when(c == 0)
        def _():
            run(u0_hbm)

        @pl.when(c == 1)
        def _():
            run(u1_hbm)

        plsc.subcore_barrier()
        sl = pl.ds(s * _RPT, _RPT)

        @pl.when(c == 0)
        def _():
            pltpu.sync_copy(acc.at[sl], o0_hbm.at[sl])

        @pl.when(c == 1)
        def _():
            pltpu.sync_copy(acc.at[sl], o1_hbm.at[sl])

    return agg_kernel(src3, dst3, u0, u1)


def _sc_agg_full(src3, dst3, g):
    cpt = src3.shape[1]
    d = g.shape[1]

    @functools.partial(
        pl.kernel,
        out_type=[jax.ShapeDtypeStruct((_NPAD, d), jnp.float32),
                  jax.ShapeDtypeStruct((_NPAD, d), jnp.float32)],
        mesh=_sc_mesh(),
        scratch_types=[
            pltpu.VMEM((cpt, _K), jnp.int32),
            pltpu.VMEM((cpt, _K), jnp.int32),
            pltpu.VMEM((_K, d), jnp.float32),
            pltpu.VMEM_SHARED((_NPAD, d), jnp.float32),
            pltpu.SemaphoreType.DMA,
        ],
        compiler_params=pltpu.CompilerParams(use_tc_tiling_on_sc=False),
    )
    def agg_kernel(src_hbm, dst_hbm, g_hbm, o0_hbm, o1_hbm,
                   src_v, dst_v, buf, acc, sem):
        c = lax.axis_index("c")
        s = lax.axis_index("s")
        w = c * _NS + s

        @pl.loop(0, _K)
        def _(i):
            @pl.loop(0, d, step=_L)
            def _(j):
                buf[i, pl.ds(j, _L)] = jnp.zeros((_L,), jnp.float32)

        for r in range(_RPT // _K):
            pltpu.sync_copy(buf, acc.at[pl.ds(s * _RPT + r * _K, _K)])
        plsc.subcore_barrier()

        pltpu.sync_copy(src_hbm.at[w], src_v)
        pltpu.sync_copy(dst_hbm.at[w], dst_v)

        @pl.loop(0, cpt)
        def _(j):
            pltpu.async_copy(g_hbm.at[src_v.at[j]], buf, sem).wait()
            pltpu.sync_copy(buf, acc.at[dst_v.at[j]], add=True)

        plsc.subcore_barrier()
        sl = pl.ds(s * _RPT, _RPT)

        @pl.when(c == 0)
        def _():
            pltpu.sync_copy(acc.at[sl], o0_hbm.at[sl])

        @pl.when(c == 1)
        def _():
            pltpu.sync_copy(acc.at[sl], o1_hbm.at[sl])

    return agg_kernel(src3, dst3, g)


def _dinv_of(d0_ref, d1_ref):
    return lax.rsqrt(d0_ref[:, 0:1] + d1_ref[:, 0:1] + 1.0)


def _tc_prep(d0, d1, x):
    bn = 2000

    def body(d0_ref, d1_ref, x_ref, u0_ref, u1_ref):
        dinv = _dinv_of(d0_ref, d1_ref)
        u = x_ref[:, :] * dinv
        u0_ref[:, :] = u[:, :128]
        u1_ref[:, :] = u[:, 128:]

    return pl.pallas_call(
        body,
        grid=(_N // bn,),
        in_specs=[pl.BlockSpec((bn, _L), lambda i: (i, 0)),
                  pl.BlockSpec((bn, _L), lambda i: (i, 0)),
                  pl.BlockSpec((bn, 256), lambda i: (i, 0))],
        out_specs=[pl.BlockSpec((bn, 128), lambda i: (i, 0)),
                   pl.BlockSpec((bn, 128), lambda i: (i, 0))],
        out_shape=[jax.ShapeDtypeStruct((_N, 128), jnp.float32),
                   jax.ShapeDtypeStruct((_N, 128), jnp.float32)],
        compiler_params=pltpu.CompilerParams(
            dimension_semantics=("parallel",)),
    )(d0, d1, x)


def _tc_mm1(d0, d1, x, s1a, s1b, w1, b1r):
    bn = 1000

    def body(d0_ref, d1_ref, x_ref, a_ref, b_ref, w_ref, bias_ref, h_ref):
        dinv = _dinv_of(d0_ref, d1_ref)
        d2 = dinv * dinv
        za = a_ref[:, :] * dinv + x_ref[:, :128] * d2
        zb = b_ref[:, :] * dinv + x_ref[:, 128:] * d2
        z = jnp.concatenate([za, zb], axis=1)
        h = lax.dot_general(z, w_ref[:, :], (((1,), (0,)), ((), ())),
                            precision=lax.Precision.HIGHEST,
                            preferred_element_type=jnp.float32)
        h_ref[:, :] = jnp.maximum(h + bias_ref[:, :], 0.0)

    return pl.pallas_call(
        body,
        grid=(_N // bn,),
        in_specs=[pl.BlockSpec((bn, _L), lambda i: (i, 0)),
                  pl.BlockSpec((bn, _L), lambda i: (i, 0)),
                  pl.BlockSpec((bn, 256), lambda i: (i, 0)),
                  pl.BlockSpec((bn, 128), lambda i: (i, 0)),
                  pl.BlockSpec((bn, 128), lambda i: (i, 0)),
                  pl.BlockSpec((256, 512), lambda i: (0, 0)),
                  pl.BlockSpec((1, 512), lambda i: (0, 0))],
        out_specs=pl.BlockSpec((bn, 512), lambda i: (i, 0)),
        out_shape=jax.ShapeDtypeStruct((_N, 512), jnp.float32),
        compiler_params=pltpu.CompilerParams(
            dimension_semantics=("parallel",)),
    )(d0, d1, x, s1a, s1b, w1, b1r)


def _tc_mm2(d0, d1, h, w2):
    bn = 2000

    def body(d0_ref, d1_ref, h_ref, w_ref, g_ref):
        dinv = _dinv_of(d0_ref, d1_ref)
        g = lax.dot_general(h_ref[:, :], w_ref[:, :], (((1,), (0,)), ((), ())),
                            precision=lax.Precision.HIGHEST,
                            preferred_element_type=jnp.float32)
        g_ref[:, :] = g * dinv

    return pl.pallas_call(
        body,
        grid=(_N // bn,),
        in_specs=[pl.BlockSpec((bn, _L), lambda i: (i, 0)),
                  pl.BlockSpec((bn, _L), lambda i: (i, 0)),
                  pl.BlockSpec((bn, 512), lambda i: (i, 0)),
                  pl.BlockSpec((512, 64), lambda i: (0, 0))],
        out_specs=pl.BlockSpec((bn, 64), lambda i: (i, 0)),
        out_shape=jax.ShapeDtypeStruct((_N, 64), jnp.float32),
        compiler_params=pltpu.CompilerParams(
            dimension_semantics=("parallel",)),
    )(d0, d1, h, w2)


def _tc_final(d0, d1, t0, t1, g, b2r):
    bn = 2000

    def body(d0_ref, d1_ref, t0_ref, t1_ref, g_ref, bias_ref, o_ref):
        dinv = _dinv_of(d0_ref, d1_ref)
        s2 = t0_ref[:, :] + t1_ref[:, :] + g_ref[:, :]
        o_ref[:, :] = s2 * dinv + bias_ref[:, :]

    return pl.pallas_call(
        body,
        grid=(_N // bn,),
        in_specs=[pl.BlockSpec((bn, _L), lambda i: (i, 0)),
                  pl.BlockSpec((bn, _L), lambda i: (i, 0)),
                  pl.BlockSpec((bn, 64), lambda i: (i, 0)),
                  pl.BlockSpec((bn, 64), lambda i: (i, 0)),
                  pl.BlockSpec((bn, 64), lambda i: (i, 0)),
                  pl.BlockSpec((1, 64), lambda i: (0, 0))],
        out_specs=pl.BlockSpec((bn, 64), lambda i: (i, 0)),
        out_shape=jax.ShapeDtypeStruct((_N, 64), jnp.float32),
        compiler_params=pltpu.CompilerParams(
            dimension_semantics=("parallel",)),
    )(d0, d1, t0, t1, g, b2r)


def kernel(x, edge_index, W1, b1, W2, b2):
    e = edge_index.shape[1]
    epad = -(-e // (_NW * _K)) * (_NW * _K)
    src = edge_index[0]
    dst = edge_index[1]
    if epad != e:
        src = jnp.concatenate([src, jnp.zeros((epad - e,), jnp.int32)])
        dst = jnp.concatenate([dst, jnp.full((epad - e,), _N, jnp.int32)])
    src_w = src.reshape(_NW, epad // _NW // _K, _K)
    dst_w = dst.reshape(_NW, epad // _NW // _K, _K)
    src_s = src.reshape(_NS, epad // _NS // _K, _K)
    dst_s = dst.reshape(_NS, epad // _NS // _K, _K)

    d0, d1 = _sc_degree(dst_w)
    u0, u1 = _tc_prep(d0, d1, x)
    s1a, s1b = _sc_agg_split(src_s, dst_s, u0, u1)
    h = _tc_mm1(d0, d1, x, s1a, s1b, W1, b1.reshape(1, -1))
    g = _tc_mm2(d0, d1, h, W2)
    t0, t1 = _sc_agg_full(src_w, dst_w, g)
    return _tc_final(d0, d1, t0, t1, g, b2.reshape(1, -1))
